# Initial kernel scaffold; baseline (speedup 1.0000x reference)
import jax
import jax.numpy as jnp
from jax.experimental import pallas as pl


def kernel(cat_features, dense_features, lin_table, V, W0, b0, W1, b1, W2, b2, Wfc, bfc):
    raise NotImplementedError("write your pallas kernel here")


# speedup vs baseline: 1.0473x; 1.0473x over previous
import functools

import jax
import jax.numpy as jnp
from jax import lax
from jax.experimental import pallas as pl
from jax.experimental.pallas import tpu as pltpu
from jax.experimental.pallas import tpu_sc as plsc

_NC = 2
_NS = 16
_NW = _NC * _NS


def _sc_gather(V, lin_table, idx, chunk):
    BM = idx.shape[0]
    K = V.shape[1]
    per_w = BM // _NW
    n_chunks = per_w // chunk
    assert per_w % chunk == 0 and BM % _NW == 0 and chunk % 8 == 0

    mesh = plsc.VectorSubcoreMesh(
        core_axis_name="c", subcore_axis_name="s",
        num_cores=_NC, num_subcores=_NS,
    )

    @functools.partial(
        pl.kernel,
        mesh=mesh,
        compiler_params=pltpu.CompilerParams(use_tc_tiling_on_sc=False),
        out_type=(
            jax.ShapeDtypeStruct((BM, K), jnp.float32),
            jax.ShapeDtypeStruct((BM, 1), jnp.float32),
        ),
        scratch_types=[
            pltpu.VMEM((2, chunk), jnp.int32),
            pltpu.VMEM((2, chunk, K), jnp.float32),
            pltpu.VMEM((2, chunk, 1), jnp.float32),
            pltpu.SemaphoreType.DMA,
            pltpu.SemaphoreType.DMA,
        ],
    )
    def gather_kernel(v_hbm, lin_hbm, idx_hbm, emb_out, lin_out,
                      idx_v, rows_v, linrow_v, sem_v, sem_l):
        wid = lax.axis_index("s") * _NC + lax.axis_index("c")
        base = wid * per_w

        def fire(slot, g):
            off = base + g * chunk
            pltpu.sync_copy(idx_hbm.at[pl.ds(off, chunk)], idx_v.at[slot])
            pltpu.async_copy(v_hbm.at[idx_v.at[slot]], rows_v.at[slot], sem_v)
            pltpu.async_copy(lin_hbm.at[idx_v.at[slot]], linrow_v.at[slot],
                             sem_l)

        def drain(slot, g):
            off = base + g * chunk
            pltpu.make_async_copy(v_hbm.at[idx_v.at[slot]], rows_v.at[slot],
                                  sem_v).wait()
            pltpu.make_async_copy(lin_hbm.at[idx_v.at[slot]],
                                  linrow_v.at[slot], sem_l).wait()
            pltpu.sync_copy(rows_v.at[slot], emb_out.at[pl.ds(off, chunk)])
            pltpu.sync_copy(linrow_v.at[slot], lin_out.at[pl.ds(off, chunk)])

        fire(0, 0)
        for g in range(1, n_chunks):
            fire(g % 2, g)
            drain((g - 1) % 2, g - 1)
        drain((n_chunks - 1) % 2, n_chunks - 1)

    return gather_kernel(V, lin_table, idx)


def _mlp_block(embd_ref, dense_ref, linv_ref, w0e_ref, w0d_ref, b0_ref,
               w1_ref, b1_ref, w2_ref, b2_ref, wh_ref, wli_ref, out_ref,
               *, m_fields, k_dim):
    e = embd_ref[...]
    s = e[:, 0:k_dim]
    for m in range(1, m_fields):
        s = s + e[:, m * k_dim:(m + 1) * k_dim]
    sum_sq = jnp.sum(e * e, axis=1, keepdims=True)
    sq_sum = jnp.sum(s * s, axis=1, keepdims=True)
    inter = 0.5 * (sq_sum - sum_sq)
    lin = jnp.sum(linv_ref[...], axis=1, keepdims=True)

    h = e @ w0e_ref[...] + dense_ref[...] @ w0d_ref[...] + b0_ref[...]
    h = jnp.maximum(h, 0.0)
    h = jnp.maximum(h @ w1_ref[...] + b1_ref[...], 0.0)
    h = jnp.maximum(h @ w2_ref[...] + b2_ref[...], 0.0)
    wli = wli_ref[...]
    z = (h @ wh_ref[...] + lin * wli[0, 0] + inter * wli[0, 1]
         + wli[0, 2])
    out_ref[...] = jax.nn.sigmoid(z)


def _tc_mlp(embd_flat, dense, linv, W0, b0, W1, b1, W2, b2, Wfc, bfc,
            block_b, interpret=False):
    B, MK = embd_flat.shape
    M = linv.shape[1]
    K = MK // M
    D = dense.shape[1]
    H0, H1, H2 = W0.shape[1], W1.shape[1], W2.shape[1]
    w0e = W0[:MK]
    w0d = W0[MK:]
    wh = Wfc[2:]
    wli = jnp.concatenate([Wfc[0:1, 0], Wfc[1:2, 0], bfc]).reshape(1, 3)
    grid = (B // block_b,)

    out = pl.pallas_call(
        functools.partial(_mlp_block, m_fields=M, k_dim=K),
        grid=grid,
        in_specs=[
            pl.BlockSpec((block_b, MK), lambda i: (i, 0)),
            pl.BlockSpec((block_b, D), lambda i: (i, 0)),
            pl.BlockSpec((block_b, M), lambda i: (i, 0)),
            pl.BlockSpec((MK, H0), lambda i: (0, 0)),
            pl.BlockSpec((D, H0), lambda i: (0, 0)),
            pl.BlockSpec((1, H0), lambda i: (0, 0)),
            pl.BlockSpec((H0, H1), lambda i: (0, 0)),
            pl.BlockSpec((1, H1), lambda i: (0, 0)),
            pl.BlockSpec((H1, H2), lambda i: (0, 0)),
            pl.BlockSpec((1, H2), lambda i: (0, 0)),
            pl.BlockSpec((H2, 1), lambda i: (0, 0)),
            pl.BlockSpec((1, 3), lambda i: (0, 0)),
        ],
        out_specs=pl.BlockSpec((block_b, 1), lambda i: (i, 0)),
        out_shape=jax.ShapeDtypeStruct((B, 1), jnp.float32),
        interpret=interpret,
    )(embd_flat, dense, linv, w0e, w0d, b0.reshape(1, H0), W1,
      b1.reshape(1, H1), W2, b2.reshape(1, H2), wh, wli)
    return out[:, 0]


def kernel(cat_features, dense_features, lin_table, V, W0, b0, W1, b1,
           W2, b2, Wfc, bfc):
    B, M = cat_features.shape
    K = V.shape[1]
    idx = cat_features.reshape(-1).astype(jnp.int32)
    emb_rows, lin_rows = _sc_gather(V, lin_table, idx, chunk=256)
    embd_flat = emb_rows.reshape(B, M * K)
    linv = lin_rows.reshape(B, M)
    return _tc_mlp(embd_flat, dense_features, linv, W0, b0, W1, b1, W2,
                   b2, Wfc, bfc, block_b=512)


# speedup vs baseline: 1.8684x; 1.7840x over previous
import functools

import jax
import jax.numpy as jnp
from jax import lax
from jax.experimental import pallas as pl
from jax.experimental.pallas import tpu as pltpu
from jax.experimental.pallas import tpu_sc as plsc

_NC = 2
_NS = 16
_NW = _NC * _NS


def _sc_gather(V, lin_table, idx, chunk=256, nbuf=3):
    BM = idx.shape[0]
    K = V.shape[1]
    lin_flat = lin_table.reshape(-1)
    per_w = BM // _NW
    n_chunks = per_w // chunk
    prefire = nbuf - 1
    assert BM % _NW == 0 and per_w % chunk == 0 and chunk % 8 == 0
    idx2d = idx.reshape(_NW * n_chunks, chunk)

    mesh = plsc.VectorSubcoreMesh(
        core_axis_name="c", subcore_axis_name="s",
        num_cores=_NC, num_subcores=_NS,
    )

    @functools.partial(
        pl.kernel,
        mesh=mesh,
        compiler_params=pltpu.CompilerParams(use_tc_tiling_on_sc=False),
        out_type=(
            jax.ShapeDtypeStruct((BM, K), jnp.float32),
            jax.ShapeDtypeStruct((_NW, n_chunks, chunk), jnp.float32),
        ),
        scratch_types=[
            pltpu.VMEM((n_chunks, chunk), jnp.int32),
            pltpu.VMEM((n_chunks, chunk), jnp.float32),
            pltpu.VMEM((nbuf, chunk, K), jnp.float32),
            pltpu.SemaphoreType.DMA,
            pltpu.SemaphoreType.DMA,
            pltpu.SemaphoreType.DMA,
        ],
    )
    def gather_kernel(v_hbm, lin_hbm, idx_hbm, emb_out, lin_out,
                      idx_v, lin_v, rows_v, sem_g, sem_o, sem_l):
        wid = lax.axis_index("s") * _NC + lax.axis_index("c")
        base = wid * per_w
        pltpu.sync_copy(idx_hbm.at[pl.ds(wid * n_chunks, n_chunks)], idx_v)
        for g in range(n_chunks):
            pltpu.async_copy(lin_hbm.at[idx_v.at[g]], lin_v.at[g], sem_l)

        def fire(g):
            pltpu.async_copy(v_hbm.at[idx_v.at[g]], rows_v.at[g % nbuf],
                             sem_g)

        def wait_gather(g):
            pltpu.make_async_copy(v_hbm.at[idx_v.at[g]],
                                  rows_v.at[g % nbuf], sem_g).wait()

        def copy_out(g):
            pltpu.async_copy(rows_v.at[g % nbuf],
                             emb_out.at[pl.ds(base + g * chunk, chunk)],
                             sem_o)

        def wait_out(g):
            pltpu.make_async_copy(rows_v.at[g % nbuf],
                                  emb_out.at[pl.ds(base + g * chunk, chunk)],
                                  sem_o).wait()

        for g in range(prefire):
            fire(g)
        for g in range(n_chunks):
            wait_gather(g)
            copy_out(g)
            f = g + prefire
            if f < n_chunks:
                if f >= nbuf:
                    wait_out(f - nbuf)
                fire(f)
        for g in range(n_chunks - nbuf, n_chunks):
            wait_out(g)
        for g in range(n_chunks):
            pltpu.make_async_copy(lin_hbm.at[idx_v.at[g]], lin_v.at[g],
                                  sem_l).wait()
        pltpu.sync_copy(lin_v, lin_out.at[wid])

    return gather_kernel(V, lin_flat, idx2d)


def _mlp_block(embd_ref, dense_ref, linv_ref, w0e_ref, w0d_ref, b0_ref,
               w1_ref, b1_ref, w2_ref, b2_ref, wh_ref, wli_ref, out_ref,
               *, m_fields, k_dim):
    e = embd_ref[...]
    s = e[:, 0:k_dim]
    for m in range(1, m_fields):
        s = s + e[:, m * k_dim:(m + 1) * k_dim]
    sum_sq = jnp.sum(e * e, axis=1, keepdims=True)
    sq_sum = jnp.sum(s * s, axis=1, keepdims=True)
    inter = 0.5 * (sq_sum - sum_sq)
    lin = jnp.sum(linv_ref[...], axis=1, keepdims=True)

    h = e @ w0e_ref[...] + dense_ref[...] @ w0d_ref[...] + b0_ref[...]
    h = jnp.maximum(h, 0.0)
    h = jnp.maximum(h @ w1_ref[...] + b1_ref[...], 0.0)
    h = jnp.maximum(h @ w2_ref[...] + b2_ref[...], 0.0)
    wli = wli_ref[...]
    z = (h @ wh_ref[...] + lin * wli[0, 0] + inter * wli[0, 1]
         + wli[0, 2])
    out_ref[...] = jax.nn.sigmoid(z)


def _tc_mlp(embd_flat, dense, linv, W0, b0, W1, b1, W2, b2, Wfc, bfc,
            block_b, interpret=False):
    B, MK = embd_flat.shape
    M = linv.shape[1]
    K = MK // M
    D = dense.shape[1]
    H0, H1, H2 = W0.shape[1], W1.shape[1], W2.shape[1]
    w0e = W0[:MK]
    w0d = W0[MK:]
    wh = Wfc[2:]
    wli = jnp.concatenate([Wfc[0:1, 0], Wfc[1:2, 0], bfc]).reshape(1, 3)
    grid = (B // block_b,)

    out = pl.pallas_call(
        functools.partial(_mlp_block, m_fields=M, k_dim=K),
        grid=grid,
        in_specs=[
            pl.BlockSpec((block_b, MK), lambda i: (i, 0)),
            pl.BlockSpec((block_b, D), lambda i: (i, 0)),
            pl.BlockSpec((block_b, M), lambda i: (i, 0)),
            pl.BlockSpec((MK, H0), lambda i: (0, 0)),
            pl.BlockSpec((D, H0), lambda i: (0, 0)),
            pl.BlockSpec((1, H0), lambda i: (0, 0)),
            pl.BlockSpec((H0, H1), lambda i: (0, 0)),
            pl.BlockSpec((1, H1), lambda i: (0, 0)),
            pl.BlockSpec((H1, H2), lambda i: (0, 0)),
            pl.BlockSpec((1, H2), lambda i: (0, 0)),
            pl.BlockSpec((H2, 1), lambda i: (0, 0)),
            pl.BlockSpec((1, 3), lambda i: (0, 0)),
        ],
        out_specs=pl.BlockSpec((block_b, 1), lambda i: (i, 0)),
        out_shape=jax.ShapeDtypeStruct((B, 1), jnp.float32),
        interpret=interpret,
    )(embd_flat, dense, linv, w0e, w0d, b0.reshape(1, H0), W1,
      b1.reshape(1, H1), W2, b2.reshape(1, H2), wh, wli)
    return out[:, 0]


def kernel(cat_features, dense_features, lin_table, V, W0, b0, W1, b1,
           W2, b2, Wfc, bfc):
    B, M = cat_features.shape
    K = V.shape[1]
    idx = cat_features.reshape(-1).astype(jnp.int32)
    emb_rows, lin_rows = _sc_gather(V, lin_table, idx)
    embd_flat = emb_rows.reshape(B, M * K)
    linv = lin_rows.reshape(B, M)
    return _tc_mlp(embd_flat, dense_features, linv, W0, b0, W1, b1, W2,
                   b2, Wfc, bfc, block_b=512)
